# baseline (speedup 1.0000x reference)
import jax
import jax.numpy as jnp
from jax.experimental import pallas as pl
from jax.experimental.pallas import tpu as pltpu

_B = 1024
_LOG_B = 10


def _halve(x):
    s = x.shape[1]
    if s == 8:
        return [x]
    h = s // 2
    a = x[:, :h]
    c = x[:, h:]
    return _halve(jnp.minimum(a, c)) + _halve(jnp.maximum(a, c))


def _roll_stages(blk, strides, lio):
    b = blk.shape[0]
    for m in strides:
        isfirst = (lio & m) == 0
        lo = jnp.minimum(blk, pltpu.roll(blk, b - m, 0))
        hi = jnp.maximum(blk, pltpu.roll(blk, m, 0))
        blk = jnp.where(isfirst, lo, hi)
    return blk


def _cascade_asc(blk, lio):
    b, lanes = blk.shape
    pcs = _halve(blk[None])
    blk = jnp.concatenate(pcs, axis=1).reshape(b, lanes)
    return _roll_stages(blk, (4, 2, 1), lio)


def _phase_pure(blk, kk, lio):
    b, lanes = blk.shape
    top = 1 << (kk - 1)
    if top >= 8:
        g2 = b // (2 * top)
        pcs = _halve(blk.reshape(g2, 2 * top, lanes))
        blk = jnp.concatenate(pcs, axis=1).reshape(b, lanes)
        strides = (4, 2, 1)
    else:
        strides = tuple(1 << j for j in range(kk - 1, -1, -1))
    return _roll_stages(blk, strides, lio)


def _make_signs(lio):
    s = [jnp.where((lio & (1 << kk)) == 0, 1.0, -1.0).astype(jnp.float32)
         for kk in range(1, _LOG_B)]
    pre = s[0]
    mids = [s[k] * s[k + 1] for k in range(len(s) - 1)]
    post = s[-1]
    return pre, mids, post


def _sort_block(blk, asc, signs, lio):
    pre, mids, post = signs
    blk = blk * pre
    for kk in range(1, _LOG_B):
        blk = _phase_pure(blk, kk, lio)
        if kk < _LOG_B - 1:
            blk = blk * mids[kk - 1]
    blk = blk * post
    sg = jnp.where(asc, jnp.float32(1.0), jnp.float32(-1.0))
    blk = blk * sg
    blk = _cascade_asc(blk, lio)
    return blk * sg


def _merge_block(blk, up, lio):
    sg = jnp.where(up, jnp.float32(1.0), jnp.float32(-1.0))
    return _cascade_asc(blk * sg, lio) * sg


def _sort_inplace(ref, signs, lio):
    n = ref.shape[0]
    nblk = n // _B
    p = n.bit_length() - 1

    def body_a(i, carry):
        blk = ref[pl.ds(i * _B, _B), :]
        asc = (i & 1) == 0
        ref[pl.ds(i * _B, _B), :] = _sort_block(blk, asc, signs, lio)
        return carry

    jax.lax.fori_loop(0, nblk, body_a, 0)

    for kk in range(_LOG_B + 1, p + 1):
        for jj in range(kk - 1, _LOG_B - 1, -1):
            m = 1 << jj
            mb = m // _B

            def body_c(t, carry, mb=mb, m=m, shift=kk - jj - 1):
                g = t // mb
                r = t - g * mb
                ia = g * (2 * m) + r * _B
                ib = ia + m
                a = ref[pl.ds(ia, _B), :]
                c = ref[pl.ds(ib, _B), :]
                lo = jnp.minimum(a, c)
                hi = jnp.maximum(a, c)
                up = ((g >> shift) & 1) == 0
                ref[pl.ds(ia, _B), :] = jnp.where(up, lo, hi)
                ref[pl.ds(ib, _B), :] = jnp.where(up, hi, lo)
                return carry

            jax.lax.fori_loop(0, n // (2 * _B), body_c, 0)

        def body_b(i, carry, kk=kk):
            blk = ref[pl.ds(i * _B, _B), :]
            up = ((i >> (kk - _LOG_B)) & 1) == 0
            ref[pl.ds(i * _B, _B), :] = _merge_block(blk, up, lio)
            return carry

        jax.lax.fori_loop(0, nblk, body_b, 0)


def _qag_kernel(p_hbm, t_hbm, o_ref, pbuf, tbuf, sem_p, sem_t):
    cp_p = pltpu.make_async_copy(p_hbm, pbuf, sem_p)
    cp_t = pltpu.make_async_copy(t_hbm, tbuf, sem_t)
    cp_p.start()
    cp_t.start()
    lio = jax.lax.broadcasted_iota(jnp.int32, (_B, 1), 0)
    signs = _make_signs(lio)
    cp_p.wait()
    _sort_inplace(pbuf, signs, lio)
    cp_t.wait()
    _sort_inplace(tbuf, signs, lio)

    n = pbuf.shape[0]
    mse_b = min(2048, n)

    def mse_body(i, acc):
        d = pbuf[pl.ds(i * mse_b, mse_b), :] - tbuf[pl.ds(i * mse_b, mse_b), :]
        return acc + jnp.sum(d * d)

    o_ref[0, 0] = jax.lax.fori_loop(0, n // mse_b, mse_body, jnp.float32(0.0))


@jax.jit
def kernel(pred, target):
    rows, n = pred.shape
    out = pl.pallas_call(
        _qag_kernel,
        out_shape=jax.ShapeDtypeStruct((1, 1), jnp.float32),
        in_specs=[
            pl.BlockSpec(memory_space=pl.ANY),
            pl.BlockSpec(memory_space=pl.ANY),
        ],
        out_specs=pl.BlockSpec(memory_space=pltpu.SMEM),
        scratch_shapes=[
            pltpu.VMEM((n, rows), jnp.float32),
            pltpu.VMEM((n, rows), jnp.float32),
            pltpu.SemaphoreType.DMA,
            pltpu.SemaphoreType.DMA,
        ],
    )(pred.T, target.T)
    return out[0, 0] / (rows * n)


# speedup vs baseline: 1.0087x; 1.0087x over previous
import jax
import jax.numpy as jnp
from jax.experimental import pallas as pl
from jax.experimental.pallas import tpu as pltpu

_B = 512
_LOG_B = 9


def _halve(x):
    s = x.shape[1]
    if s == 8:
        return [x]
    h = s // 2
    a = x[:, :h]
    c = x[:, h:]
    return _halve(jnp.minimum(a, c)) + _halve(jnp.maximum(a, c))


def _roll_stages(blk, strides, lio):
    b = blk.shape[0]
    for m in strides:
        isfirst = (lio & m) == 0
        lo = jnp.minimum(blk, pltpu.roll(blk, b - m, 0))
        hi = jnp.maximum(blk, pltpu.roll(blk, m, 0))
        blk = jnp.where(isfirst, lo, hi)
    return blk


def _cascade_asc(blk, lio):
    b, lanes = blk.shape
    pcs = _halve(blk[None])
    blk = jnp.concatenate(pcs, axis=1).reshape(b, lanes)
    return _roll_stages(blk, (4, 2, 1), lio)


def _phase_pure(blk, kk, lio):
    b, lanes = blk.shape
    top = 1 << (kk - 1)
    if top >= 8:
        g2 = b // (2 * top)
        pcs = _halve(blk.reshape(g2, 2 * top, lanes))
        blk = jnp.concatenate(pcs, axis=1).reshape(b, lanes)
        strides = (4, 2, 1)
    else:
        strides = tuple(1 << j for j in range(kk - 1, -1, -1))
    return _roll_stages(blk, strides, lio)


def _make_signs(lio):
    s = [jnp.where((lio & (1 << kk)) == 0, 1.0, -1.0).astype(jnp.float32)
         for kk in range(1, _LOG_B)]
    pre = s[0]
    mids = [s[k] * s[k + 1] for k in range(len(s) - 1)]
    post = s[-1]
    return pre, mids, post


def _sort_block(blk, asc, signs, lio):
    pre, mids, post = signs
    blk = blk * pre
    for kk in range(1, _LOG_B):
        blk = _phase_pure(blk, kk, lio)
        if kk < _LOG_B - 1:
            blk = blk * mids[kk - 1]
    blk = blk * post
    sg = jnp.where(asc, jnp.float32(1.0), jnp.float32(-1.0))
    blk = blk * sg
    blk = _cascade_asc(blk, lio)
    return blk * sg


def _merge_block(blk, up, lio):
    sg = jnp.where(up, jnp.float32(1.0), jnp.float32(-1.0))
    return _cascade_asc(blk * sg, lio) * sg


def _sort_inplace(refs, signs, lio):
    n = refs[0].shape[0]
    nblk = n // _B
    p = n.bit_length() - 1

    def body_a(i, carry):
        asc = (i & 1) == 0
        for ref in refs:
            blk = ref[pl.ds(i * _B, _B), :]
            ref[pl.ds(i * _B, _B), :] = _sort_block(blk, asc, signs, lio)
        return carry

    jax.lax.fori_loop(0, nblk, body_a, 0)

    for kk in range(_LOG_B + 1, p + 1):
        for jj in range(kk - 1, _LOG_B - 1, -1):
            m = 1 << jj
            mb = m // _B

            def body_c(t, carry, mb=mb, m=m, shift=kk - jj - 1):
                g = t // mb
                r = t - g * mb
                ia = g * (2 * m) + r * _B
                ib = ia + m
                up = ((g >> shift) & 1) == 0
                for ref in refs:
                    a = ref[pl.ds(ia, _B), :]
                    c = ref[pl.ds(ib, _B), :]
                    lo = jnp.minimum(a, c)
                    hi = jnp.maximum(a, c)
                    ref[pl.ds(ia, _B), :] = jnp.where(up, lo, hi)
                    ref[pl.ds(ib, _B), :] = jnp.where(up, hi, lo)
                return carry

            jax.lax.fori_loop(0, n // (2 * _B), body_c, 0)

        def body_b(i, carry, kk=kk):
            up = ((i >> (kk - _LOG_B)) & 1) == 0
            for ref in refs:
                blk = ref[pl.ds(i * _B, _B), :]
                ref[pl.ds(i * _B, _B), :] = _merge_block(blk, up, lio)
            return carry

        jax.lax.fori_loop(0, nblk, body_b, 0)


def _qag_kernel(p_hbm, t_hbm, o_ref, pbuf, tbuf, sem_p, sem_t):
    cp_p = pltpu.make_async_copy(p_hbm, pbuf, sem_p)
    cp_t = pltpu.make_async_copy(t_hbm, tbuf, sem_t)
    cp_p.start()
    cp_t.start()
    lio = jax.lax.broadcasted_iota(jnp.int32, (_B, 1), 0)
    signs = _make_signs(lio)
    cp_p.wait()
    cp_t.wait()
    _sort_inplace((pbuf, tbuf), signs, lio)

    n = pbuf.shape[0]
    mse_b = min(2048, n)

    def mse_body(i, acc):
        d = pbuf[pl.ds(i * mse_b, mse_b), :] - tbuf[pl.ds(i * mse_b, mse_b), :]
        return acc + jnp.sum(d * d)

    o_ref[0, 0] = jax.lax.fori_loop(0, n // mse_b, mse_body, jnp.float32(0.0))


@jax.jit
def kernel(pred, target):
    rows, n = pred.shape
    out = pl.pallas_call(
        _qag_kernel,
        out_shape=jax.ShapeDtypeStruct((1, 1), jnp.float32),
        in_specs=[
            pl.BlockSpec(memory_space=pl.ANY),
            pl.BlockSpec(memory_space=pl.ANY),
        ],
        out_specs=pl.BlockSpec(memory_space=pltpu.SMEM),
        scratch_shapes=[
            pltpu.VMEM((n, rows), jnp.float32),
            pltpu.VMEM((n, rows), jnp.float32),
            pltpu.SemaphoreType.DMA,
            pltpu.SemaphoreType.DMA,
        ],
    )(pred.T, target.T)
    return out[0, 0] / (rows * n)


# speedup vs baseline: 1.0238x; 1.0149x over previous
import jax
import jax.numpy as jnp
from jax.experimental import pallas as pl
from jax.experimental.pallas import tpu as pltpu

_B = 512
_LOG_B = 9


def _halve(x):
    s = x.shape[1]
    if s == 8:
        return [x]
    h = s // 2
    a = x[:, :h]
    c = x[:, h:]
    return _halve(jnp.minimum(a, c)) + _halve(jnp.maximum(a, c))


def _roll_stages(blk, strides, lio):
    b = blk.shape[0]
    for m in strides:
        isfirst = (lio & m) == 0
        lo = jnp.minimum(blk, pltpu.roll(blk, b - m, 0))
        hi = jnp.maximum(blk, pltpu.roll(blk, m, 0))
        blk = jnp.where(isfirst, lo, hi)
    return blk


def _cascade_asc(blk, lio):
    b, lanes = blk.shape
    pcs = _halve(blk[None])
    blk = jnp.concatenate(pcs, axis=1).reshape(b, lanes)
    return _roll_stages(blk, (4, 2, 1), lio)


def _phase_pure(blk, kk, lio):
    b, lanes = blk.shape
    top = 1 << (kk - 1)
    if top >= 8:
        g2 = b // (2 * top)
        pcs = _halve(blk.reshape(g2, 2 * top, lanes))
        blk = jnp.concatenate(pcs, axis=1).reshape(b, lanes)
        strides = (4, 2, 1)
    else:
        strides = tuple(1 << j for j in range(kk - 1, -1, -1))
    return _roll_stages(blk, strides, lio)


def _make_signs(lio):
    s = [jnp.where((lio & (1 << kk)) == 0, 1.0, -1.0).astype(jnp.float32)
         for kk in range(1, _LOG_B)]
    pre = s[0]
    mids = [s[k] * s[k + 1] for k in range(len(s) - 1)]
    post = s[-1]
    return pre, mids, post


def _sort_block(blk, asc, signs, lio):
    pre, mids, post = signs
    blk = blk * pre
    for kk in range(1, _LOG_B):
        blk = _phase_pure(blk, kk, lio)
        if kk < _LOG_B - 1:
            blk = blk * mids[kk - 1]
    blk = blk * post
    sg = jnp.where(asc, jnp.float32(1.0), jnp.float32(-1.0))
    blk = blk * sg
    blk = _cascade_asc(blk, lio)
    return blk * sg


def _merge_block(blk, up, lio):
    sg = jnp.where(up, jnp.float32(1.0), jnp.float32(-1.0))
    return _cascade_asc(blk * sg, lio) * sg


def _sort_inplace(ref, signs, lio):
    n = ref.shape[0]
    nblk = n // _B
    p = n.bit_length() - 1

    def body_a(i, carry):
        blk = ref[pl.ds(i * _B, _B), :]
        asc = (i & 1) == 0
        ref[pl.ds(i * _B, _B), :] = _sort_block(blk, asc, signs, lio)
        return carry

    jax.lax.fori_loop(0, nblk, body_a, 0)

    for kk in range(_LOG_B + 1, p + 1):
        for jj in range(kk - 1, _LOG_B, -1):
            m = 1 << jj
            mb = m // _B

            def body_c(t, carry, mb=mb, m=m, shift=kk - jj - 1):
                g = t // mb
                r = t - g * mb
                ia = g * (2 * m) + r * _B
                ib = ia + m
                a = ref[pl.ds(ia, _B), :]
                c = ref[pl.ds(ib, _B), :]
                lo = jnp.minimum(a, c)
                hi = jnp.maximum(a, c)
                up = ((g >> shift) & 1) == 0
                ref[pl.ds(ia, _B), :] = jnp.where(up, lo, hi)
                ref[pl.ds(ib, _B), :] = jnp.where(up, hi, lo)
                return carry

            jax.lax.fori_loop(0, n // (2 * _B), body_c, 0)

        def body_b(g, carry, kk=kk):
            ia = 2 * g * _B
            ib = ia + _B
            up = ((g >> (kk - _LOG_B - 1)) & 1) == 0
            sg = jnp.where(up, jnp.float32(1.0), jnp.float32(-1.0))
            a = ref[pl.ds(ia, _B), :] * sg
            c = ref[pl.ds(ib, _B), :] * sg
            na = jnp.minimum(a, c)
            nb = jnp.maximum(a, c)
            ref[pl.ds(ia, _B), :] = _cascade_asc(na, lio) * sg
            ref[pl.ds(ib, _B), :] = _cascade_asc(nb, lio) * sg
            return carry

        jax.lax.fori_loop(0, nblk // 2, body_b, 0)


def _qag_kernel(p_hbm, t_hbm, o_ref, pbuf, tbuf, sem_p, sem_t):
    cp_p = pltpu.make_async_copy(p_hbm, pbuf, sem_p)
    cp_t = pltpu.make_async_copy(t_hbm, tbuf, sem_t)
    cp_p.start()
    cp_t.start()
    lio = jax.lax.broadcasted_iota(jnp.int32, (_B, 1), 0)
    signs = _make_signs(lio)
    cp_p.wait()
    _sort_inplace(pbuf, signs, lio)
    cp_t.wait()
    _sort_inplace(tbuf, signs, lio)

    n = pbuf.shape[0]
    mse_b = min(2048, n)

    def mse_body(i, acc):
        d = pbuf[pl.ds(i * mse_b, mse_b), :] - tbuf[pl.ds(i * mse_b, mse_b), :]
        return acc + jnp.sum(d * d)

    o_ref[0, 0] = jax.lax.fori_loop(0, n // mse_b, mse_body, jnp.float32(0.0))


@jax.jit
def kernel(pred, target):
    rows, n = pred.shape
    out = pl.pallas_call(
        _qag_kernel,
        out_shape=jax.ShapeDtypeStruct((1, 1), jnp.float32),
        in_specs=[
            pl.BlockSpec(memory_space=pl.ANY),
            pl.BlockSpec(memory_space=pl.ANY),
        ],
        out_specs=pl.BlockSpec(memory_space=pltpu.SMEM),
        scratch_shapes=[
            pltpu.VMEM((n, rows), jnp.float32),
            pltpu.VMEM((n, rows), jnp.float32),
            pltpu.SemaphoreType.DMA,
            pltpu.SemaphoreType.DMA,
        ],
    )(pred.T, target.T)
    return out[0, 0] / (rows * n)
